# baseline (speedup 1.0000x reference)
import functools

import jax
import jax.numpy as jnp
from jax import lax
from jax.experimental import pallas as pl
from jax.experimental.pallas import tpu as pltpu
from jax.experimental.pallas import tpu_sc as plsc

B = 8
T_PHN = 512
H = 256
MEL = 2048
LANES = 16

NW = 32
ROWS_PER_W = (B * MEL) // NW
CHUNK = 32
NBUF = 12
NCHUNK = ROWS_PER_W // CHUNK
ZROW = B * T_PHN
W_PER_B = MEL // ROWS_PER_W


def _lr_body(x, dur, out, dur_v, idx2, *ring):
    bufs = ring[:NBUF]
    gsems = ring[NBUF:2 * NBUF]
    ssems = ring[2 * NBUF:3 * NBUF]

    cid = lax.axis_index("c")
    sid = lax.axis_index("s")
    wid = sid * 2 + cid
    base = wid * ROWS_PER_W
    b = wid // W_PER_B

    pltpu.sync_copy(dur.at[pl.ds(base, ROWS_PER_W)], dur_v.at[pl.ds(0, ROWS_PER_W)])
    xb = x.at[b]
    zero16 = jnp.zeros((LANES,), jnp.int32)
    lanes16 = lax.iota(jnp.int32, LANES)
    gdnums = lax.GatherDimensionNumbers(
        offset_dims=(), collapsed_slice_dims=(0,), start_index_map=(0,))

    def _lane_min(v):
        for s in (8, 4, 2, 1):
            perm = jnp.bitwise_xor(lanes16, s)
            shuf = lax.gather(v, perm[:, None], gdnums, slice_sizes=(1,),
                              mode=lax.GatherScatterMode.PROMISE_IN_BOUNDS)
            v = jnp.minimum(v, shuf)
        return v

    zany = []
    for c in range(NCHUNK):
        row = idx2.at[c]
        dmin = jnp.full((LANES,), 1, jnp.int32)
        for j in range(CHUNK // LANES):
            d = dur_v[pl.ds(c * CHUNK + j * LANES, LANES)]
            row[pl.ds(j * LANES, LANES)] = jnp.maximum(d - 1, zero16)
            dmin = jnp.minimum(dmin, d)
        zany.append(_lane_min(dmin)[0] == 0)

    gh = [None] * NCHUNK
    sh = [None] * NCHUNK
    for c in range(NBUF):
        gh[c] = pltpu.async_copy(xb.at[idx2.at[c]], bufs[c], gsems[c])

    zrow = jnp.zeros((LANES,), jnp.float32)

    for c in range(NCHUNK):
        slot = c % NBUF
        gh[c].wait()

        @pl.when(zany[c])
        def _fix(c=c, slot=slot):
            def fix_body(r, carry):
                dvec = dur_v[pl.ds(c * CHUNK + r, LANES)]

                @pl.when(dvec[0] == 0)
                def _z():
                    brow = bufs[slot].at[r]
                    for k in range(H // LANES):
                        brow[pl.ds(k * LANES, LANES)] = zrow
                return carry

            lax.fori_loop(0, CHUNK, fix_body, 0)

        sh[c] = pltpu.async_copy(
            bufs[slot], out.at[pl.ds(base + c * CHUNK, CHUNK)], ssems[slot])
        nxt = c + NBUF
        if nxt < NCHUNK:
            sh[c].wait()
            gh[nxt] = pltpu.async_copy(xb.at[idx2.at[nxt]], bufs[slot], gsems[slot])
    for c in range(NCHUNK - NBUF, NCHUNK):
        sh[c].wait()


_lr_call = pl.kernel(
    _lr_body,
    out_type=jax.ShapeDtypeStruct((B * MEL, H), jnp.float32),
    mesh=plsc.VectorSubcoreMesh(core_axis_name="c", subcore_axis_name="s"),
    scratch_types=(
        pltpu.VMEM((ROWS_PER_W + LANES,), jnp.int32),
        pltpu.VMEM((NCHUNK, CHUNK), jnp.int32),
        *([pltpu.VMEM((CHUNK, H), jnp.float32)] * NBUF),
        *([pltpu.SemaphoreType.DMA] * NBUF),
        *([pltpu.SemaphoreType.DMA] * NBUF),
    ),
)


def _mel_tc_body(dur_ref, mel_ref):
    d = dur_ref[...]
    t = lax.broadcasted_iota(jnp.int32, (B, MEL), 1)
    mel_ref[...] = jnp.min(jnp.where(d == 0, t, MEL), axis=1)


_mel_tc = pl.pallas_call(
    _mel_tc_body,
    out_shape=jax.ShapeDtypeStruct((B,), jnp.int32),
)


def kernel(x, duration):
    dur2 = duration.astype(jnp.int32)
    out_flat = _lr_call(x, dur2.reshape(B * MEL))
    return out_flat.reshape(B, MEL, H), _mel_tc(dur2)


# speedup vs baseline: 1.0208x; 1.0208x over previous
import functools

import jax
import jax.numpy as jnp
from jax import lax
from jax.experimental import pallas as pl
from jax.experimental.pallas import tpu as pltpu
from jax.experimental.pallas import tpu_sc as plsc

B = 8
T_PHN = 512
H = 256
MEL = 2048
LANES = 16

NW = 32
ROWS_PER_W = (B * MEL) // NW
CHUNK = 64
NBUF = 6
NCHUNK = ROWS_PER_W // CHUNK
ZROW = B * T_PHN
W_PER_B = MEL // ROWS_PER_W


def _lr_body(x, dur, out, dur_v, idx2,
             b0, b1, b2, b3, b4, b5, g0, g1, g2, g3, g4, g5,
             s0, s1, s2, s3, s4, s5):
    bufs = (b0, b1, b2, b3, b4, b5)
    gsems = (g0, g1, g2, g3, g4, g5)
    ssems = (s0, s1, s2, s3, s4, s5)

    cid = lax.axis_index("c")
    sid = lax.axis_index("s")
    wid = sid * 2 + cid
    base = wid * ROWS_PER_W
    b = wid // W_PER_B

    pltpu.sync_copy(dur.at[pl.ds(base, ROWS_PER_W)], dur_v.at[pl.ds(0, ROWS_PER_W)])
    xb = x.at[b]
    zero16 = jnp.zeros((LANES,), jnp.int32)
    lanes16 = lax.iota(jnp.int32, LANES)
    gdnums = lax.GatherDimensionNumbers(
        offset_dims=(), collapsed_slice_dims=(0,), start_index_map=(0,))

    def _lane_min(v):
        for s in (8, 4, 2, 1):
            perm = jnp.bitwise_xor(lanes16, s)
            shuf = lax.gather(v, perm[:, None], gdnums, slice_sizes=(1,),
                              mode=lax.GatherScatterMode.PROMISE_IN_BOUNDS)
            v = jnp.minimum(v, shuf)
        return v

    zany = []
    for c in range(NCHUNK):
        row = idx2.at[c]
        dmin = jnp.full((LANES,), 1, jnp.int32)
        for j in range(CHUNK // LANES):
            d = dur_v[pl.ds(c * CHUNK + j * LANES, LANES)]
            row[pl.ds(j * LANES, LANES)] = jnp.maximum(d - 1, zero16)
            dmin = jnp.minimum(dmin, d)
        zany.append(_lane_min(dmin)[0] == 0)

    gh = [None] * NCHUNK
    sh = [None] * NCHUNK
    for c in range(NBUF):
        gh[c] = pltpu.async_copy(xb.at[idx2.at[c]], bufs[c], gsems[c])

    zrow = jnp.zeros((LANES,), jnp.float32)

    for c in range(NCHUNK):
        slot = c % NBUF
        gh[c].wait()

        @pl.when(zany[c])
        def _fix(c=c, slot=slot):
            def fix_body(r, carry):
                dvec = dur_v[pl.ds(c * CHUNK + r, LANES)]

                @pl.when(dvec[0] == 0)
                def _z():
                    brow = bufs[slot].at[r]
                    for k in range(H // LANES):
                        brow[pl.ds(k * LANES, LANES)] = zrow
                return carry

            lax.fori_loop(0, CHUNK, fix_body, 0)

        sh[c] = pltpu.async_copy(
            bufs[slot], out.at[pl.ds(base + c * CHUNK, CHUNK)], ssems[slot])
        nxt = c + NBUF
        if nxt < NCHUNK:
            sh[c].wait()
            gh[nxt] = pltpu.async_copy(xb.at[idx2.at[nxt]], bufs[slot], gsems[slot])
    for c in range(NCHUNK - NBUF, NCHUNK):
        sh[c].wait()


_lr_call = pl.kernel(
    _lr_body,
    out_type=jax.ShapeDtypeStruct((B * MEL, H), jnp.float32),
    mesh=plsc.VectorSubcoreMesh(core_axis_name="c", subcore_axis_name="s"),
    scratch_types=(
        pltpu.VMEM((ROWS_PER_W + LANES,), jnp.int32),
        pltpu.VMEM((NCHUNK, CHUNK), jnp.int32),
        pltpu.VMEM((CHUNK, H), jnp.float32),
        pltpu.VMEM((CHUNK, H), jnp.float32),
        pltpu.VMEM((CHUNK, H), jnp.float32),
        pltpu.VMEM((CHUNK, H), jnp.float32),
        pltpu.VMEM((CHUNK, H), jnp.float32),
        pltpu.VMEM((CHUNK, H), jnp.float32),
        pltpu.SemaphoreType.DMA,
        pltpu.SemaphoreType.DMA,
        pltpu.SemaphoreType.DMA,
        pltpu.SemaphoreType.DMA,
        pltpu.SemaphoreType.DMA,
        pltpu.SemaphoreType.DMA,
        pltpu.SemaphoreType.DMA,
        pltpu.SemaphoreType.DMA,
        pltpu.SemaphoreType.DMA,
        pltpu.SemaphoreType.DMA,
        pltpu.SemaphoreType.DMA,
        pltpu.SemaphoreType.DMA,
    ),
)


def _mel_tc_body(dur_ref, mel_ref):
    d = dur_ref[...]
    t = lax.broadcasted_iota(jnp.int32, (B, MEL), 1)
    mel_ref[...] = jnp.min(jnp.where(d == 0, t, MEL), axis=1)


_mel_tc = pl.pallas_call(
    _mel_tc_body,
    out_shape=jax.ShapeDtypeStruct((B,), jnp.int32),
)


def kernel(x, duration):
    dur2 = duration.astype(jnp.int32)
    out_flat = _lr_call(x, dur2.reshape(B * MEL))
    return out_flat.reshape(B, MEL, H), _mel_tc(dur2)
